# Initial kernel scaffold; baseline (speedup 1.0000x reference)
import jax
import jax.numpy as jnp
from jax.experimental import pallas as pl


def kernel(predicted_weights, target_weights, edge_index_for_similarity, node_features_for_similarity):
    raise NotImplementedError("write your pallas kernel here")


# speedup vs baseline: 1.3442x; 1.3442x over previous
import functools

import jax
import jax.numpy as jnp
from jax import lax
from jax.experimental import pallas as pl
from jax.experimental.pallas import tpu as pltpu
from jax.experimental.pallas import tpu_sc as plsc

SIGMA = 1.0
NC = 2
NS = 16
NW = NC * NS
L = 16
C = 80
NBUF = 5
JU = 8


@functools.cache
def _build(E, N, D):
    assert E % NW == 0 and D % L == 0
    epw = E // NW
    assert epw % C == 0
    nchunk = epw // C
    assert nchunk % NBUF == 0
    mesh = plsc.VectorSubcoreMesh(core_axis_name="c", subcore_axis_name="s")

    @functools.partial(
        pl.kernel,
        out_type=jax.ShapeDtypeStruct((NW * 5 * L,), jnp.float32),
        mesh=mesh,
        compiler_params=pltpu.CompilerParams(needs_layout_passes=False),
        scratch_types=[
            pltpu.VMEM((epw,), jnp.int32),
            pltpu.VMEM((epw,), jnp.int32),
            *[pltpu.VMEM((C, D), jnp.float32) for _ in range(NBUF)],
            *[pltpu.VMEM((C, D), jnp.float32) for _ in range(NBUF)],
            pltpu.VMEM((NBUF * 2 * C,), jnp.float32),
            pltpu.VMEM((5 * L,), jnp.float32),
            pltpu.SemaphoreType.DMA,
            pltpu.SemaphoreType.DMA,
            pltpu.SemaphoreType.DMA,
            pltpu.SemaphoreType.DMA,
            pltpu.SemaphoreType.DMA,
        ],
    )
    def body(row_hbm, col_hbm, pred_hbm, tgt_hbm, feat_hbm, out_hbm,
             idx_row_v, idx_col_v,
             rb0, rb1, rb2, rb3, rb4, cb0, cb1, cb2, cb3, cb4,
             auxbuf, out_stage, sem0, sem1, sem2, sem3, sem4):
        rowbufs = [rb0, rb1, rb2, rb3, rb4]
        colbufs = [cb0, cb1, cb2, cb3, cb4]
        sems = [sem0, sem1, sem2, sem3, sem4]
        wid = lax.axis_index("c") * NS + lax.axis_index("s")
        base = wid * epw

        pltpu.sync_copy(row_hbm.at[pl.ds(base, epw)], idx_row_v)
        pltpu.sync_copy(col_hbm.at[pl.ds(base, epw)], idx_col_v)

        def copies(c, b):
            off = base + c * C
            return [
                (feat_hbm.at[idx_row_v.at[pl.ds(c * C, C)]], rowbufs[b]),
                (feat_hbm.at[idx_col_v.at[pl.ds(c * C, C)]], colbufs[b]),
                (pred_hbm.at[pl.ds(off, C)], auxbuf.at[pl.ds(b * 2 * C, C)]),
                (tgt_hbm.at[pl.ds(off, C)], auxbuf.at[pl.ds(b * 2 * C + C, C)]),
            ]

        def issue(c, b):
            for src, dst in copies(c, b):
                pltpu.async_copy(src, dst, sems[b])

        def drain(c, b):
            for src, dst in copies(c, b):
                pltpu.make_async_copy(src, dst, sems[b]).wait()

        def compute(b, carry):
            s_neg, s_pos, s_one, n_ones, n_zeros = carry
            rb = rowbufs[b]
            cb = colbufs[b]
            for g in range(C // L):
                e0 = g * L
                eidx = e0 + lax.iota(jnp.int32, L)

                def jstep(j, acc, eidx=eidx, rb=rb, cb=cb):
                    for u in range(JU):
                        jv = jnp.full((L,), j * JU + u, jnp.int32)
                        diff = (plsc.load_gather(rb, [eidx, jv])
                                - plsc.load_gather(cb, [eidx, jv]))
                        acc = acc + diff * diff
                    return acc

                dist = lax.fori_loop(0, D // JU, jstep,
                                     jnp.zeros((L,), jnp.float32))
                dist = dist * (1.0 / (SIGMA * SIGMA))
                p16 = auxbuf[pl.ds(b * 2 * C + e0, L)]
                t16 = auxbuf[pl.ds(b * 2 * C + C + e0, L)]
                lv = (p16 - t16) * (p16 - t16)
                s_neg = s_neg + jnp.exp(-dist) * lv
                s_pos = s_pos + jnp.exp(dist) * lv
                s_one = s_one + lv
                n_ones = n_ones + jnp.where(t16 == 1.0, 1.0, 0.0)
                n_zeros = n_zeros + jnp.where(t16 == 0.0, 1.0, 0.0)
            return (s_neg, s_pos, s_one, n_ones, n_zeros)

        for b in range(NBUF):
            issue(b, b)

        def outer(i, carry):
            for b in range(NBUF):
                c = i * NBUF + b
                drain(c, b)
                carry = compute(b, carry)

                @pl.when(i < nchunk // NBUF - 1)
                def _(c=c, b=b):
                    issue(c + NBUF, b)
            return carry

        zero = jnp.zeros((L,), jnp.float32)
        acc = lax.fori_loop(0, nchunk // NBUF, outer,
                            (zero, zero, zero, zero, zero))
        for k in range(5):
            out_stage[pl.ds(k * L, L)] = acc[k]
        pltpu.sync_copy(out_stage, out_hbm.at[pl.ds(wid * 5 * L, 5 * L)])

    return body


def kernel(predicted_weights, target_weights, edge_index_for_similarity,
           node_features_for_similarity):
    E = predicted_weights.shape[0]
    N, D = node_features_for_similarity.shape
    row = edge_index_for_similarity[0].astype(jnp.int32)
    col = edge_index_for_similarity[1].astype(jnp.int32)
    pred = predicted_weights.astype(jnp.float32)
    tgt = target_weights.astype(jnp.float32)
    feat = node_features_for_similarity.astype(jnp.float32)

    parts = _build(E, N, D)(row, col, pred, tgt, feat).reshape(NW, 5 * L)

    s_neg = jnp.sum(parts[:, 0 * L:1 * L])
    s_pos = jnp.sum(parts[:, 1 * L:2 * L])
    s_one = jnp.sum(parts[:, 2 * L:3 * L])
    n_ones = jnp.sum(parts[:, 3 * L:4 * L])
    n_zeros = jnp.sum(parts[:, 4 * L:5 * L])
    all_ones = n_ones == float(E)
    all_zeros = n_zeros == float(E)
    if SIGMA > 1e-07:
        total = jnp.where(all_ones, s_neg,
                          jnp.where(all_zeros, s_pos, s_one))
    else:
        total = s_one
    return total / E


# speedup vs baseline: 2.4911x; 1.8532x over previous
import functools

import jax
import jax.numpy as jnp
from jax import lax
from jax.experimental import pallas as pl
from jax.experimental.pallas import tpu as pltpu
from jax.experimental.pallas import tpu_sc as plsc

SIGMA = 1.0
NC = 2
NS = 16
NW = NC * NS
L = 16
C = 80
ND = 4
JU = 8


@functools.cache
def _build(E, N, D):
    assert E % NW == 0 and D % L == 0
    epw = E // NW
    assert epw % C == 0
    nchunk = epw // C
    assert (nchunk - 1) % ND == 0
    mesh = plsc.VectorSubcoreMesh(core_axis_name="c", subcore_axis_name="s")

    @functools.partial(
        pl.kernel,
        out_type=jax.ShapeDtypeStruct((NW * 5 * L,), jnp.float32),
        mesh=mesh,
        compiler_params=pltpu.CompilerParams(needs_layout_passes=False),
        scratch_types=[
            pltpu.VMEM_SHARED((N, D), jnp.float32),
            *[pltpu.VMEM((C, D), jnp.float32) for _ in range(ND)],
            *[pltpu.VMEM((C,), jnp.int32) for _ in range(ND)],
            *[pltpu.VMEM((C,), jnp.int32) for _ in range(ND)],
            *[pltpu.VMEM((C,), jnp.float32) for _ in range(ND)],
            *[pltpu.VMEM((C,), jnp.float32) for _ in range(ND)],
            pltpu.VMEM((5 * L,), jnp.float32),
            *[pltpu.SemaphoreType.DMA for _ in range(2 * ND)],
        ],
    )
    def body(row_hbm, col_hbm, pred_hbm, tgt_hbm, feat_hbm, negfeat_hbm,
             out_hbm, shared_feat, *scratch):
        fbuf = scratch[0:ND]
        ridx = scratch[ND:2 * ND]
        cidx = scratch[2 * ND:3 * ND]
        paux = scratch[3 * ND:4 * ND]
        taux = scratch[4 * ND:5 * ND]
        out_stage = scratch[5 * ND]
        isem = scratch[5 * ND + 1:5 * ND + 1 + ND]
        fsem = scratch[5 * ND + 1 + ND:5 * ND + 1 + 2 * ND]

        wid = lax.axis_index("c") * NS + lax.axis_index("s")
        base = wid * epw

        sid = lax.axis_index("s")
        rpt = (N // NS) // 8 * 8

        @pl.when(sid < NS - 1)
        def _():
            pltpu.sync_copy(feat_hbm.at[pl.ds(sid * rpt, rpt)],
                            shared_feat.at[pl.ds(sid * rpt, rpt)])

        @pl.when(sid == NS - 1)
        def _():
            last = N - (NS - 1) * rpt
            pltpu.sync_copy(feat_hbm.at[pl.ds((NS - 1) * rpt, last)],
                            shared_feat.at[pl.ds((NS - 1) * rpt, last)])

        plsc.subcore_barrier()

        def idx_copies(c, k):
            off = base + c * C
            return [
                (row_hbm.at[pl.ds(off, C)], ridx[k]),
                (col_hbm.at[pl.ds(off, C)], cidx[k]),
                (pred_hbm.at[pl.ds(off, C)], paux[k]),
                (tgt_hbm.at[pl.ds(off, C)], taux[k]),
            ]

        def phase_a(c, k):
            for src, dst in idx_copies(c, k):
                pltpu.async_copy(src, dst, isem[k])

        def phase_b1(c, k):
            for src, dst in idx_copies(c, k):
                pltpu.make_async_copy(src, dst, isem[k]).wait()
            pltpu.async_copy(shared_feat.at[ridx[k]], fbuf[k], fsem[k])

        def phase_b2(c, k):
            pltpu.make_async_copy(shared_feat.at[ridx[k]], fbuf[k],
                                  fsem[k]).wait()
            pltpu.async_copy(negfeat_hbm.at[cidx[k]], fbuf[k], fsem[k],
                             add=True)

        def phase_c(c, k, carry):
            pltpu.make_async_copy(negfeat_hbm.at[cidx[k]], fbuf[k],
                                  fsem[k]).wait()
            s_neg, s_pos, s_one, n_ones, n_zeros = carry
            fb = fbuf[k]
            for g in range(C // L):
                e0 = g * L
                eidx = e0 + lax.iota(jnp.int32, L)

                def jstep(j, acc, eidx=eidx, fb=fb):
                    for u in range(JU):
                        jv = jnp.full((L,), j * JU + u, jnp.int32)
                        diff = plsc.load_gather(fb, [eidx, jv])
                        acc = acc + diff * diff
                    return acc

                dist = lax.fori_loop(0, D // JU, jstep,
                                     jnp.zeros((L,), jnp.float32))
                dist = dist * (1.0 / (SIGMA * SIGMA))
                p16 = paux[k][pl.ds(e0, L)]
                t16 = taux[k][pl.ds(e0, L)]
                lv = (p16 - t16) * (p16 - t16)
                s_neg = s_neg + jnp.exp(-dist) * lv
                s_pos = s_pos + jnp.exp(dist) * lv
                s_one = s_one + lv
                n_ones = n_ones + jnp.where(t16 == 1.0, 1.0, 0.0)
                n_zeros = n_zeros + jnp.where(t16 == 0.0, 1.0, 0.0)
            return (s_neg, s_pos, s_one, n_ones, n_zeros)

        phase_a(0, 0)
        phase_b1(0, 0)
        phase_a(1, 1)
        phase_b2(0, 0)
        phase_b1(1, 1)
        phase_a(2, 2)

        def outer(i, carry):
            for b in range(ND):
                s = i * ND + b + 3
                carry = phase_c(s - 3, b, carry)
                phase_b2(s - 2, (b + 1) % ND)

                @pl.when(s <= nchunk)
                def _(s=s, b=b):
                    phase_b1(s - 1, (b + 2) % ND)

                @pl.when(s < nchunk)
                def _(s=s, b=b):
                    phase_a(s, (b + 3) % ND)
            return carry

        zero = jnp.zeros((L,), jnp.float32)
        acc = lax.fori_loop(0, (nchunk - 1) // ND, outer,
                            (zero, zero, zero, zero, zero))
        acc = phase_c(nchunk - 1, (nchunk - 1) % ND, acc)
        for k in range(5):
            out_stage[pl.ds(k * L, L)] = acc[k]
        pltpu.sync_copy(out_stage, out_hbm.at[pl.ds(wid * 5 * L, 5 * L)])

    return body


def kernel(predicted_weights, target_weights, edge_index_for_similarity,
           node_features_for_similarity):
    E = predicted_weights.shape[0]
    N, D = node_features_for_similarity.shape
    row = edge_index_for_similarity[0].astype(jnp.int32)
    col = edge_index_for_similarity[1].astype(jnp.int32)
    pred = predicted_weights.astype(jnp.float32)
    tgt = target_weights.astype(jnp.float32)
    feat = node_features_for_similarity.astype(jnp.float32)
    negfeat = -feat

    parts = _build(E, N, D)(row, col, pred, tgt, feat,
                            negfeat).reshape(NW, 5 * L)

    s_neg = jnp.sum(parts[:, 0 * L:1 * L])
    s_pos = jnp.sum(parts[:, 1 * L:2 * L])
    s_one = jnp.sum(parts[:, 2 * L:3 * L])
    n_ones = jnp.sum(parts[:, 3 * L:4 * L])
    n_zeros = jnp.sum(parts[:, 4 * L:5 * L])
    all_ones = n_ones == float(E)
    all_zeros = n_zeros == float(E)
    if SIGMA > 1e-07:
        total = jnp.where(all_ones, s_neg,
                          jnp.where(all_zeros, s_pos, s_one))
    else:
        total = s_one
    return total / E


# speedup vs baseline: 6.1334x; 2.4621x over previous
import functools

import jax
import jax.numpy as jnp
from jax import lax
from jax.experimental import pallas as pl
from jax.experimental.pallas import tpu as pltpu
from jax.experimental.pallas import tpu_sc as plsc

SIGMA = 1.0
NC = 2
NS = 16
NW = NC * NS
L = 16
C = 80
ND = 4
JU = 8


@functools.cache
def _build(E, N, D):
    assert E % NW == 0 and D % L == 0
    epw = E // NW
    assert epw % C == 0
    nchunk = epw // C
    assert (nchunk - 1) % ND == 0
    mesh = plsc.VectorSubcoreMesh(core_axis_name="c", subcore_axis_name="s")

    @functools.partial(
        pl.kernel,
        out_type=jax.ShapeDtypeStruct((NW * 5 * L,), jnp.float32),
        mesh=mesh,
        compiler_params=pltpu.CompilerParams(needs_layout_passes=False),
        scratch_types=[
            pltpu.VMEM_SHARED((N, D), jnp.float32),
            *[pltpu.VMEM((C, D), jnp.float32) for _ in range(ND)],
            *[pltpu.VMEM((C,), jnp.int32) for _ in range(ND)],
            *[pltpu.VMEM((C,), jnp.int32) for _ in range(ND)],
            *[pltpu.VMEM((C,), jnp.float32) for _ in range(ND)],
            *[pltpu.VMEM((C,), jnp.float32) for _ in range(ND)],
            pltpu.VMEM((5 * L,), jnp.float32),
            *[pltpu.SemaphoreType.DMA for _ in range(2 * ND)],
        ],
    )
    def body(row_hbm, col_hbm, pred_hbm, tgt_hbm, feat_hbm, negfeat_hbm,
             out_hbm, shared_feat, *scratch):
        fbuf = scratch[0:ND]
        ridx = scratch[ND:2 * ND]
        cidx = scratch[2 * ND:3 * ND]
        paux = scratch[3 * ND:4 * ND]
        taux = scratch[4 * ND:5 * ND]
        out_stage = scratch[5 * ND]
        isem = scratch[5 * ND + 1:5 * ND + 1 + ND]
        fsem = scratch[5 * ND + 1 + ND:5 * ND + 1 + 2 * ND]

        wid = lax.axis_index("c") * NS + lax.axis_index("s")
        base = wid * epw

        sid = lax.axis_index("s")
        rpt = (N // NS) // 8 * 8

        @pl.when(sid < NS - 1)
        def _():
            pltpu.sync_copy(feat_hbm.at[pl.ds(sid * rpt, rpt)],
                            shared_feat.at[pl.ds(sid * rpt, rpt)])

        @pl.when(sid == NS - 1)
        def _():
            last = N - (NS - 1) * rpt
            pltpu.sync_copy(feat_hbm.at[pl.ds((NS - 1) * rpt, last)],
                            shared_feat.at[pl.ds((NS - 1) * rpt, last)])

        plsc.subcore_barrier()

        def idx_copies(c, k):
            off = base + c * C
            return [
                (row_hbm.at[pl.ds(off, C)], ridx[k]),
                (col_hbm.at[pl.ds(off, C)], cidx[k]),
                (pred_hbm.at[pl.ds(off, C)], paux[k]),
                (tgt_hbm.at[pl.ds(off, C)], taux[k]),
            ]

        def phase_a(c, k):
            for src, dst in idx_copies(c, k):
                pltpu.async_copy(src, dst, isem[k])

        def phase_b1(c, k):
            for src, dst in idx_copies(c, k):
                pltpu.make_async_copy(src, dst, isem[k]).wait()
            pltpu.async_copy(shared_feat.at[ridx[k]], fbuf[k], fsem[k])

        def phase_b2(c, k):
            pltpu.make_async_copy(shared_feat.at[ridx[k]], fbuf[k],
                                  fsem[k]).wait()
            pltpu.async_copy(negfeat_hbm.at[cidx[k]], fbuf[k], fsem[k],
                             add=True)

        def phase_c(c, k, carry):
            pltpu.make_async_copy(negfeat_hbm.at[cidx[k]], fbuf[k],
                                  fsem[k]).wait()
            s_neg, s_pos, s_one, n_ones, n_zeros = carry
            fb = fbuf[k]
            lane = lax.iota(jnp.int32, L)
            for g in range(C // L):
                e0 = g * L
                eidx = e0 + lane

                def jstep(j, acc, eidx=eidx, fb=fb):
                    for u in range(JU):
                        jv = (jnp.full((L,), j * JU + u, jnp.int32)
                              + lane) & (D - 1)
                        diff = plsc.load_gather(fb, [eidx, jv])
                        acc = acc + diff * diff
                    return acc

                dist = lax.fori_loop(0, D // JU, jstep,
                                     jnp.zeros((L,), jnp.float32))
                dist = dist * (1.0 / (SIGMA * SIGMA))
                p16 = paux[k][pl.ds(e0, L)]
                t16 = taux[k][pl.ds(e0, L)]
                lv = (p16 - t16) * (p16 - t16)
                s_neg = s_neg + jnp.exp(-dist) * lv
                s_pos = s_pos + jnp.exp(dist) * lv
                s_one = s_one + lv
                n_ones = n_ones + jnp.where(t16 == 1.0, 1.0, 0.0)
                n_zeros = n_zeros + jnp.where(t16 == 0.0, 1.0, 0.0)
            return (s_neg, s_pos, s_one, n_ones, n_zeros)

        phase_a(0, 0)
        phase_b1(0, 0)
        phase_a(1, 1)
        phase_b2(0, 0)
        phase_b1(1, 1)
        phase_a(2, 2)

        def outer(i, carry):
            for b in range(ND):
                s = i * ND + b + 3
                carry = phase_c(s - 3, b, carry)
                phase_b2(s - 2, (b + 1) % ND)

                @pl.when(s <= nchunk)
                def _(s=s, b=b):
                    phase_b1(s - 1, (b + 2) % ND)

                @pl.when(s < nchunk)
                def _(s=s, b=b):
                    phase_a(s, (b + 3) % ND)
            return carry

        zero = jnp.zeros((L,), jnp.float32)
        acc = lax.fori_loop(0, (nchunk - 1) // ND, outer,
                            (zero, zero, zero, zero, zero))
        acc = phase_c(nchunk - 1, (nchunk - 1) % ND, acc)
        for k in range(5):
            out_stage[pl.ds(k * L, L)] = acc[k]
        pltpu.sync_copy(out_stage, out_hbm.at[pl.ds(wid * 5 * L, 5 * L)])

    return body


def kernel(predicted_weights, target_weights, edge_index_for_similarity,
           node_features_for_similarity):
    E = predicted_weights.shape[0]
    N, D = node_features_for_similarity.shape
    row = edge_index_for_similarity[0].astype(jnp.int32)
    col = edge_index_for_similarity[1].astype(jnp.int32)
    pred = predicted_weights.astype(jnp.float32)
    tgt = target_weights.astype(jnp.float32)
    feat = node_features_for_similarity.astype(jnp.float32)
    negfeat = -feat

    parts = _build(E, N, D)(row, col, pred, tgt, feat,
                            negfeat).reshape(NW, 5 * L)

    s_neg = jnp.sum(parts[:, 0 * L:1 * L])
    s_pos = jnp.sum(parts[:, 1 * L:2 * L])
    s_one = jnp.sum(parts[:, 2 * L:3 * L])
    n_ones = jnp.sum(parts[:, 3 * L:4 * L])
    n_zeros = jnp.sum(parts[:, 4 * L:5 * L])
    all_ones = n_ones == float(E)
    all_zeros = n_zeros == float(E)
    if SIGMA > 1e-07:
        total = jnp.where(all_ones, s_neg,
                          jnp.where(all_zeros, s_pos, s_one))
    else:
        total = s_one
    return total / E


# speedup vs baseline: 6.3290x; 1.0319x over previous
import functools

import jax
import jax.numpy as jnp
from jax import lax
from jax.experimental import pallas as pl
from jax.experimental.pallas import tpu as pltpu
from jax.experimental.pallas import tpu_sc as plsc

SIGMA = 1.0
NC = 2
NS = 16
NW = NC * NS
L = 16
C = 80
ND = 4
JU = 8


@functools.cache
def _build(E, N, D):
    assert E % NW == 0 and D % L == 0
    epw = E // NW
    assert epw % C == 0
    nchunk = epw // C
    assert (nchunk - 1) % ND == 0
    mesh = plsc.VectorSubcoreMesh(core_axis_name="c", subcore_axis_name="s")

    @functools.partial(
        pl.kernel,
        out_type=jax.ShapeDtypeStruct((NW * 5 * L,), jnp.float32),
        mesh=mesh,
        compiler_params=pltpu.CompilerParams(needs_layout_passes=False),
        scratch_types=[
            pltpu.VMEM_SHARED((N, D), jnp.float32),
            *[pltpu.VMEM((C, D), jnp.float32) for _ in range(ND)],
            *[pltpu.VMEM((C,), jnp.int32) for _ in range(ND)],
            *[pltpu.VMEM((C,), jnp.int32) for _ in range(ND)],
            *[pltpu.VMEM((C,), jnp.float32) for _ in range(ND)],
            *[pltpu.VMEM((C,), jnp.float32) for _ in range(ND)],
            pltpu.VMEM((5 * L,), jnp.float32),
            *[pltpu.SemaphoreType.DMA for _ in range(2 * ND)],
        ],
    )
    def body(row_hbm, col_hbm, pred_hbm, tgt_hbm, feat_hbm, negfeat_hbm,
             out_hbm, shared_feat, *scratch):
        fbuf = scratch[0:ND]
        ridx = scratch[ND:2 * ND]
        cidx = scratch[2 * ND:3 * ND]
        paux = scratch[3 * ND:4 * ND]
        taux = scratch[4 * ND:5 * ND]
        out_stage = scratch[5 * ND]
        isem = scratch[5 * ND + 1:5 * ND + 1 + ND]
        fsem = scratch[5 * ND + 1 + ND:5 * ND + 1 + 2 * ND]

        wid = lax.axis_index("c") * NS + lax.axis_index("s")
        base = wid * epw

        sid = lax.axis_index("s")
        rpt = (N // NS) // 8 * 8

        @pl.when(sid < NS - 1)
        def _():
            pltpu.sync_copy(feat_hbm.at[pl.ds(sid * rpt, rpt)],
                            shared_feat.at[pl.ds(sid * rpt, rpt)])

        @pl.when(sid == NS - 1)
        def _():
            last = N - (NS - 1) * rpt
            pltpu.sync_copy(feat_hbm.at[pl.ds((NS - 1) * rpt, last)],
                            shared_feat.at[pl.ds((NS - 1) * rpt, last)])

        plsc.subcore_barrier()

        def idx_copies(c, k):
            off = base + c * C
            return [
                (row_hbm.at[pl.ds(off, C)], ridx[k]),
                (col_hbm.at[pl.ds(off, C)], cidx[k]),
                (pred_hbm.at[pl.ds(off, C)], paux[k]),
                (tgt_hbm.at[pl.ds(off, C)], taux[k]),
            ]

        def phase_a(c, k):
            for src, dst in idx_copies(c, k):
                pltpu.async_copy(src, dst, isem[k])

        def phase_b1(c, k):
            for src, dst in idx_copies(c, k):
                pltpu.make_async_copy(src, dst, isem[k]).wait()
            pltpu.async_copy(shared_feat.at[ridx[k]], fbuf[k], fsem[k])

        def phase_b2(c, k):
            pltpu.make_async_copy(shared_feat.at[ridx[k]], fbuf[k],
                                  fsem[k]).wait()
            pltpu.async_copy(negfeat_hbm.at[cidx[k]], fbuf[k], fsem[k],
                             add=True)

        def phase_c(c, k, carry):
            pltpu.make_async_copy(negfeat_hbm.at[cidx[k]], fbuf[k],
                                  fsem[k]).wait()
            s_neg, s_pos, s_one, n_ones, n_zeros = carry
            fb = fbuf[k]
            lane = lax.iota(jnp.int32, L)
            for g in range(C // L):
                e0 = g * L
                eidx = e0 + lane

                def jstep(j, accs, eidx=eidx, fb=fb):
                    accs = list(accs)
                    for u in range(JU):
                        jv = (jnp.full((L,), j * JU + u, jnp.int32)
                              + lane) & (D - 1)
                        diff = plsc.load_gather(fb, [eidx, jv])
                        accs[u % 4] = accs[u % 4] + diff * diff
                    return tuple(accs)

                z = jnp.zeros((L,), jnp.float32)
                a0, a1, a2, a3 = lax.fori_loop(0, D // JU, jstep,
                                               (z, z, z, z))
                dist = (a0 + a1) + (a2 + a3)
                dist = dist * (1.0 / (SIGMA * SIGMA))
                p16 = paux[k][pl.ds(e0, L)]
                t16 = taux[k][pl.ds(e0, L)]
                lv = (p16 - t16) * (p16 - t16)
                s_neg = s_neg + jnp.exp(-dist) * lv
                s_pos = s_pos + jnp.exp(dist) * lv
                s_one = s_one + lv
                n_ones = n_ones + jnp.where(t16 == 1.0, 1.0, 0.0)
                n_zeros = n_zeros + jnp.where(t16 == 0.0, 1.0, 0.0)
            return (s_neg, s_pos, s_one, n_ones, n_zeros)

        phase_a(0, 0)
        phase_b1(0, 0)
        phase_a(1, 1)
        phase_b2(0, 0)
        phase_b1(1, 1)
        phase_a(2, 2)

        def outer(i, carry):
            for b in range(ND):
                s = i * ND + b + 3
                carry = phase_c(s - 3, b, carry)
                phase_b2(s - 2, (b + 1) % ND)

                @pl.when(s <= nchunk)
                def _(s=s, b=b):
                    phase_b1(s - 1, (b + 2) % ND)

                @pl.when(s < nchunk)
                def _(s=s, b=b):
                    phase_a(s, (b + 3) % ND)
            return carry

        zero = jnp.zeros((L,), jnp.float32)
        acc = lax.fori_loop(0, (nchunk - 1) // ND, outer,
                            (zero, zero, zero, zero, zero))
        acc = phase_c(nchunk - 1, (nchunk - 1) % ND, acc)
        for k in range(5):
            out_stage[pl.ds(k * L, L)] = acc[k]
        pltpu.sync_copy(out_stage, out_hbm.at[pl.ds(wid * 5 * L, 5 * L)])

    return body


def kernel(predicted_weights, target_weights, edge_index_for_similarity,
           node_features_for_similarity):
    E = predicted_weights.shape[0]
    N, D = node_features_for_similarity.shape
    row = edge_index_for_similarity[0].astype(jnp.int32)
    col = edge_index_for_similarity[1].astype(jnp.int32)
    pred = predicted_weights.astype(jnp.float32)
    tgt = target_weights.astype(jnp.float32)
    feat = node_features_for_similarity.astype(jnp.float32)
    negfeat = -feat

    parts = _build(E, N, D)(row, col, pred, tgt, feat,
                            negfeat).reshape(NW, 5 * L)

    s_neg = jnp.sum(parts[:, 0 * L:1 * L])
    s_pos = jnp.sum(parts[:, 1 * L:2 * L])
    s_one = jnp.sum(parts[:, 2 * L:3 * L])
    n_ones = jnp.sum(parts[:, 3 * L:4 * L])
    n_zeros = jnp.sum(parts[:, 4 * L:5 * L])
    all_ones = n_ones == float(E)
    all_zeros = n_zeros == float(E)
    if SIGMA > 1e-07:
        total = jnp.where(all_ones, s_neg,
                          jnp.where(all_zeros, s_pos, s_one))
    else:
        total = s_one
    return total / E


# speedup vs baseline: 6.3694x; 1.0064x over previous
import functools

import jax
import jax.numpy as jnp
from jax import lax
from jax.experimental import pallas as pl
from jax.experimental.pallas import tpu as pltpu
from jax.experimental.pallas import tpu_sc as plsc

SIGMA = 1.0
NC = 2
NS = 16
NW = NC * NS
L = 16
C = 80
ND = 4
JU = 16


@functools.cache
def _build(E, N, D):
    assert E % NW == 0 and D % L == 0
    epw = E // NW
    assert epw % C == 0
    nchunk = epw // C
    assert (nchunk - 1) % ND == 0
    mesh = plsc.VectorSubcoreMesh(core_axis_name="c", subcore_axis_name="s")

    @functools.partial(
        pl.kernel,
        out_type=jax.ShapeDtypeStruct((NW * 5 * L,), jnp.float32),
        mesh=mesh,
        compiler_params=pltpu.CompilerParams(needs_layout_passes=False),
        scratch_types=[
            pltpu.VMEM_SHARED((N, D), jnp.float32),
            *[pltpu.VMEM((C, D), jnp.float32) for _ in range(ND)],
            *[pltpu.VMEM((C,), jnp.int32) for _ in range(ND)],
            *[pltpu.VMEM((C,), jnp.int32) for _ in range(ND)],
            *[pltpu.VMEM((C,), jnp.float32) for _ in range(ND)],
            *[pltpu.VMEM((C,), jnp.float32) for _ in range(ND)],
            pltpu.VMEM((5 * L,), jnp.float32),
            *[pltpu.SemaphoreType.DMA for _ in range(2 * ND)],
        ],
    )
    def body(row_hbm, col_hbm, pred_hbm, tgt_hbm, feat_hbm, negfeat_hbm,
             out_hbm, shared_feat, *scratch):
        fbuf = scratch[0:ND]
        ridx = scratch[ND:2 * ND]
        cidx = scratch[2 * ND:3 * ND]
        paux = scratch[3 * ND:4 * ND]
        taux = scratch[4 * ND:5 * ND]
        out_stage = scratch[5 * ND]
        isem = scratch[5 * ND + 1:5 * ND + 1 + ND]
        fsem = scratch[5 * ND + 1 + ND:5 * ND + 1 + 2 * ND]

        wid = lax.axis_index("c") * NS + lax.axis_index("s")
        base = wid * epw

        sid = lax.axis_index("s")
        rpt = (N // NS) // 8 * 8

        @pl.when(sid < NS - 1)
        def _():
            pltpu.sync_copy(feat_hbm.at[pl.ds(sid * rpt, rpt)],
                            shared_feat.at[pl.ds(sid * rpt, rpt)])

        @pl.when(sid == NS - 1)
        def _():
            last = N - (NS - 1) * rpt
            pltpu.sync_copy(feat_hbm.at[pl.ds((NS - 1) * rpt, last)],
                            shared_feat.at[pl.ds((NS - 1) * rpt, last)])

        plsc.subcore_barrier()

        def idx_copies(c, k):
            off = base + c * C
            return [
                (row_hbm.at[pl.ds(off, C)], ridx[k]),
                (col_hbm.at[pl.ds(off, C)], cidx[k]),
                (pred_hbm.at[pl.ds(off, C)], paux[k]),
                (tgt_hbm.at[pl.ds(off, C)], taux[k]),
            ]

        def phase_a(c, k):
            for src, dst in idx_copies(c, k):
                pltpu.async_copy(src, dst, isem[k])

        def phase_b1(c, k):
            for src, dst in idx_copies(c, k):
                pltpu.make_async_copy(src, dst, isem[k]).wait()
            pltpu.async_copy(shared_feat.at[ridx[k]], fbuf[k], fsem[k])

        def phase_b2(c, k):
            pltpu.make_async_copy(shared_feat.at[ridx[k]], fbuf[k],
                                  fsem[k]).wait()
            pltpu.async_copy(negfeat_hbm.at[cidx[k]], fbuf[k], fsem[k],
                             add=True)

        def phase_c(c, k, carry):
            pltpu.make_async_copy(negfeat_hbm.at[cidx[k]], fbuf[k],
                                  fsem[k]).wait()
            s_neg, s_pos, s_one, n_ones, n_zeros = carry
            fb = fbuf[k]
            lane = lax.iota(jnp.int32, L)
            lane9 = lane * 9
            for g in range(C // L):
                e0 = g * L
                eidx = e0 + lane

                def jstep(j, accs, eidx=eidx, fb=fb):
                    accs = list(accs)
                    for u in range(JU):
                        jv = (jnp.full((L,), j * JU + u, jnp.int32)
                              + lane9) & (D - 1)
                        diff = plsc.load_gather(fb, [eidx, jv])
                        accs[u % 4] = accs[u % 4] + diff * diff
                    return tuple(accs)

                z = jnp.zeros((L,), jnp.float32)
                a0, a1, a2, a3 = lax.fori_loop(0, D // JU, jstep,
                                               (z, z, z, z))
                dist = (a0 + a1) + (a2 + a3)
                dist = dist * (1.0 / (SIGMA * SIGMA))
                p16 = paux[k][pl.ds(e0, L)]
                t16 = taux[k][pl.ds(e0, L)]
                lv = (p16 - t16) * (p16 - t16)
                s_neg = s_neg + jnp.exp(-dist) * lv
                s_pos = s_pos + jnp.exp(dist) * lv
                s_one = s_one + lv
                n_ones = n_ones + jnp.where(t16 == 1.0, 1.0, 0.0)
                n_zeros = n_zeros + jnp.where(t16 == 0.0, 1.0, 0.0)
            return (s_neg, s_pos, s_one, n_ones, n_zeros)

        phase_a(0, 0)
        phase_b1(0, 0)
        phase_a(1, 1)
        phase_b2(0, 0)
        phase_b1(1, 1)
        phase_a(2, 2)

        def outer(i, carry):
            for b in range(ND):
                s = i * ND + b + 3
                carry = phase_c(s - 3, b, carry)
                phase_b2(s - 2, (b + 1) % ND)

                @pl.when(s <= nchunk)
                def _(s=s, b=b):
                    phase_b1(s - 1, (b + 2) % ND)

                @pl.when(s < nchunk)
                def _(s=s, b=b):
                    phase_a(s, (b + 3) % ND)
            return carry

        zero = jnp.zeros((L,), jnp.float32)
        acc = lax.fori_loop(0, (nchunk - 1) // ND, outer,
                            (zero, zero, zero, zero, zero))
        acc = phase_c(nchunk - 1, (nchunk - 1) % ND, acc)
        for k in range(5):
            out_stage[pl.ds(k * L, L)] = acc[k]
        pltpu.sync_copy(out_stage, out_hbm.at[pl.ds(wid * 5 * L, 5 * L)])

    return body


def kernel(predicted_weights, target_weights, edge_index_for_similarity,
           node_features_for_similarity):
    E = predicted_weights.shape[0]
    N, D = node_features_for_similarity.shape
    row = edge_index_for_similarity[0].astype(jnp.int32)
    col = edge_index_for_similarity[1].astype(jnp.int32)
    pred = predicted_weights.astype(jnp.float32)
    tgt = target_weights.astype(jnp.float32)
    feat = node_features_for_similarity.astype(jnp.float32)
    negfeat = -feat

    parts = _build(E, N, D)(row, col, pred, tgt, feat,
                            negfeat).reshape(NW, 5 * L)

    s_neg = jnp.sum(parts[:, 0 * L:1 * L])
    s_pos = jnp.sum(parts[:, 1 * L:2 * L])
    s_one = jnp.sum(parts[:, 2 * L:3 * L])
    n_ones = jnp.sum(parts[:, 3 * L:4 * L])
    n_zeros = jnp.sum(parts[:, 4 * L:5 * L])
    all_ones = n_ones == float(E)
    all_zeros = n_zeros == float(E)
    if SIGMA > 1e-07:
        total = jnp.where(all_ones, s_neg,
                          jnp.where(all_zeros, s_pos, s_one))
    else:
        total = s_one
    return total / E


# speedup vs baseline: 9.8317x; 1.5436x over previous
import functools

import jax
import jax.numpy as jnp
from jax import lax
from jax.experimental import pallas as pl
from jax.experimental.pallas import tpu as pltpu
from jax.experimental.pallas import tpu_sc as plsc

SIGMA = 1.0
NC = 2
NS = 16
NW = NC * NS
L = 16
C = 80
NF = 4
NI = 8
JU = 16


@functools.cache
def _build(E, N, D):
    assert E % NW == 0 and D % L == 0
    epw = E // NW
    assert epw % C == 0
    nchunk = epw // C
    assert (nchunk - 5) % NI == 0 and nchunk >= 10
    mesh = plsc.VectorSubcoreMesh(core_axis_name="c", subcore_axis_name="s")

    @functools.partial(
        pl.kernel,
        out_type=jax.ShapeDtypeStruct((NW * 5 * L,), jnp.float32),
        mesh=mesh,
        compiler_params=pltpu.CompilerParams(needs_layout_passes=False),
        scratch_types=[
            pltpu.VMEM_SHARED((N, D), jnp.float32),
            *[pltpu.VMEM((C, D), jnp.float32) for _ in range(NF)],
            *[pltpu.VMEM((C,), jnp.int32) for _ in range(NI)],
            *[pltpu.VMEM((C,), jnp.int32) for _ in range(NI)],
            *[pltpu.VMEM((C,), jnp.float32) for _ in range(NI)],
            *[pltpu.VMEM((C,), jnp.float32) for _ in range(NI)],
            pltpu.VMEM((5 * L,), jnp.float32),
            *[pltpu.SemaphoreType.DMA for _ in range(NI + NF)],
        ],
    )
    def body(row_hbm, col_hbm, pred_hbm, tgt_hbm, feat_hbm, negfeat_hbm,
             out_hbm, shared_feat, *scratch):
        o = NF
        fbuf = scratch[0:o]
        ridx = scratch[o:o + NI]
        cidx = scratch[o + NI:o + 2 * NI]
        paux = scratch[o + 2 * NI:o + 3 * NI]
        taux = scratch[o + 3 * NI:o + 4 * NI]
        out_stage = scratch[o + 4 * NI]
        isem = scratch[o + 4 * NI + 1:o + 5 * NI + 1]
        fsem = scratch[o + 5 * NI + 1:o + 5 * NI + 1 + NF]

        wid = lax.axis_index("c") * NS + lax.axis_index("s")
        base = wid * epw

        sid = lax.axis_index("s")
        rpt = (N // NS) // 8 * 8

        @pl.when(sid < NS - 1)
        def _():
            pltpu.sync_copy(feat_hbm.at[pl.ds(sid * rpt, rpt)],
                            shared_feat.at[pl.ds(sid * rpt, rpt)])

        @pl.when(sid == NS - 1)
        def _():
            last = N - (NS - 1) * rpt
            pltpu.sync_copy(feat_hbm.at[pl.ds((NS - 1) * rpt, last)],
                            shared_feat.at[pl.ds((NS - 1) * rpt, last)])

        plsc.subcore_barrier()

        def idx_copies(c, k):
            off = base + c * C
            return [
                (row_hbm.at[pl.ds(off, C)], ridx[k]),
                (col_hbm.at[pl.ds(off, C)], cidx[k]),
                (pred_hbm.at[pl.ds(off, C)], paux[k]),
                (tgt_hbm.at[pl.ds(off, C)], taux[k]),
            ]

        def phase_a(c, k):
            for src, dst in idx_copies(c, k):
                pltpu.async_copy(src, dst, isem[k])

        def phase_b1(c, k, f):
            for src, dst in idx_copies(c, k):
                pltpu.make_async_copy(src, dst, isem[k]).wait()
            pltpu.async_copy(shared_feat.at[ridx[k]], fbuf[f], fsem[f])

        def phase_b2(c, k, f):
            pltpu.make_async_copy(shared_feat.at[ridx[k]], fbuf[f],
                                  fsem[f]).wait()
            pltpu.async_copy(negfeat_hbm.at[cidx[k]], fbuf[f], fsem[f],
                             add=True)

        def phase_c(c, k, f, carry):
            pltpu.make_async_copy(negfeat_hbm.at[cidx[k]], fbuf[f],
                                  fsem[f]).wait()
            s_neg, s_pos, s_one, n_ones, n_zeros = carry
            fb = fbuf[f]
            lane = lax.iota(jnp.int32, L)
            lane9 = lane * 9
            for g in range(C // L):
                e0 = g * L
                eidx = e0 + lane

                def jstep(j, accs, eidx=eidx, fb=fb):
                    accs = list(accs)
                    for u in range(JU):
                        jv = (jnp.full((L,), j * JU + u, jnp.int32)
                              + lane9) & (D - 1)
                        diff = plsc.load_gather(fb, [eidx, jv])
                        accs[u % 4] = accs[u % 4] + diff * diff
                    return tuple(accs)

                z = jnp.zeros((L,), jnp.float32)
                a0, a1, a2, a3 = lax.fori_loop(0, D // JU, jstep,
                                               (z, z, z, z))
                dist = (a0 + a1) + (a2 + a3)
                dist = dist * (1.0 / (SIGMA * SIGMA))
                p16 = paux[k][pl.ds(e0, L)]
                t16 = taux[k][pl.ds(e0, L)]
                lv = (p16 - t16) * (p16 - t16)
                s_neg = s_neg + jnp.exp(-dist) * lv
                s_pos = s_pos + jnp.exp(dist) * lv
                s_one = s_one + lv
                n_ones = n_ones + jnp.where(t16 == 1.0, 1.0, 0.0)
                n_zeros = n_zeros + jnp.where(t16 == 0.0, 1.0, 0.0)
            return (s_neg, s_pos, s_one, n_ones, n_zeros)

        PRO = 10
        carry = tuple(jnp.zeros((L,), jnp.float32) for _ in range(5))
        for s in range(PRO):
            if s >= 5:
                c = s - 5
                carry = phase_c(c, c % NI, c % NF, carry)
            if s >= 3:
                c = s - 3
                phase_b2(c, c % NI, c % NF)
            if s >= 1:
                c = s - 1
                phase_b1(c, c % NI, c % NF)
            phase_a(s, s % NI)

        def outer(i, carry):
            for b in range(NI):
                s = i * NI + b + PRO
                c = s - 5
                carry = phase_c(c, (PRO + b - 5) % NI, (PRO + b - 5) % NF,
                                carry)

                @pl.when(s - 3 < nchunk)
                def _(s=s, b=b):
                    phase_b2(s - 3, (PRO + b - 3) % NI, (PRO + b - 3) % NF)

                @pl.when(s - 1 < nchunk)
                def _(s=s, b=b):
                    phase_b1(s - 1, (PRO + b - 1) % NI, (PRO + b - 1) % NF)

                @pl.when(s < nchunk)
                def _(s=s, b=b):
                    phase_a(s, (PRO + b) % NI)
            return carry

        acc = lax.fori_loop(0, (nchunk - 5) // NI, outer, carry)
        for k in range(5):
            out_stage[pl.ds(k * L, L)] = acc[k]
        pltpu.sync_copy(out_stage, out_hbm.at[pl.ds(wid * 5 * L, 5 * L)])

    return body


def kernel(predicted_weights, target_weights, edge_index_for_similarity,
           node_features_for_similarity):
    E = predicted_weights.shape[0]
    N, D = node_features_for_similarity.shape
    row = edge_index_for_similarity[0].astype(jnp.int32)
    col = edge_index_for_similarity[1].astype(jnp.int32)
    pred = predicted_weights.astype(jnp.float32)
    tgt = target_weights.astype(jnp.float32)
    feat = node_features_for_similarity.astype(jnp.float32)
    negfeat = -feat

    parts = _build(E, N, D)(row, col, pred, tgt, feat,
                            negfeat).reshape(NW, 5 * L)

    s_neg = jnp.sum(parts[:, 0 * L:1 * L])
    s_pos = jnp.sum(parts[:, 1 * L:2 * L])
    s_one = jnp.sum(parts[:, 2 * L:3 * L])
    n_ones = jnp.sum(parts[:, 3 * L:4 * L])
    n_zeros = jnp.sum(parts[:, 4 * L:5 * L])
    all_ones = n_ones == float(E)
    all_zeros = n_zeros == float(E)
    if SIGMA > 1e-07:
        total = jnp.where(all_ones, s_neg,
                          jnp.where(all_zeros, s_pos, s_one))
    else:
        total = s_one
    return total / E
